# baseline (speedup 1.0000x reference)
import functools

import jax
import jax.numpy as jnp
from jax import lax
from jax.experimental import pallas as pl
from jax.experimental.pallas import tpu as pltpu
from jax.experimental.pallas import tpu_sc as plsc

_CHUNK = 128
_IBLK = 80
_NBLK = 1


@functools.cache
def _make_spmm(N, D, E_pad):
    info = plsc.get_sparse_core_info()
    NC, NS = info.num_cores, info.num_subcores
    NW = NC * NS
    assert E_pad == NW * _NBLK * _IBLK * _CHUNK
    ZROWS = 8
    N_pad = -(-N // (ZROWS * NS)) * (ZROWS * NS)
    rows_per_tile = N_pad // NS
    n_z = rows_per_tile // ZROWS

    mesh = plsc.VectorSubcoreMesh(core_axis_name="c", subcore_axis_name="s")

    @functools.partial(
        pl.kernel,
        mesh=mesh,
        out_type=jax.ShapeDtypeStruct((NC, N_pad, D), jnp.float32),
        scratch_types=[
            pltpu.VMEM((_IBLK, _CHUNK), jnp.int32),
            pltpu.VMEM((_IBLK, _CHUNK), jnp.int32),
            pltpu.VMEM((_CHUNK, D), jnp.float32),
            pltpu.VMEM((ZROWS, D), jnp.float32),
            pltpu.VMEM_SHARED((N_pad, D), jnp.float32),
            pltpu.SemaphoreType.DMA,
            pltpu.SemaphoreType.DMA,
        ],
    )
    def spmm(x_hbm, src_hbm, dst_hbm, out_hbm, sidx, didx, rows,
             zbuf, acc, gsem, ssem):
        cid = lax.axis_index("c")
        sid = lax.axis_index("s")
        wid = sid * NC + cid

        for i in range(ZROWS):
            for j in range(D // 16):
                zbuf[i, pl.ds(j * 16, 16)] = jnp.zeros((16,), jnp.float32)

        def zacc(k, carry):
            pltpu.sync_copy(zbuf, acc.at[pl.ds(sid * rows_per_tile + k * ZROWS, ZROWS)])
            return carry

        lax.fori_loop(0, n_z, zacc, 0)
        plsc.subcore_barrier()

        def g_start(c):
            pltpu.make_async_copy(x_hbm.at[sidx.at[c]], rows, gsem).start()

        def g_wait(c):
            pltpu.make_async_copy(x_hbm.at[sidx.at[c]], rows, gsem).wait()

        def s_start(c):
            pltpu.async_copy(rows, acc.at[didx.at[c]], ssem, add=True)

        def s_wait(c):
            pltpu.make_async_copy(rows, acc.at[didx.at[c]], ssem).wait()

        for r in range(_NBLK):
            pltpu.sync_copy(src_hbm.at[wid, pl.ds(r * _IBLK, _IBLK)], sidx)
            pltpu.sync_copy(dst_hbm.at[wid, pl.ds(r * _IBLK, _IBLK)], didx)
            def pipe(c, carry):
                g_start(c)
                g_wait(c)
                pltpu.sync_copy(rows, acc.at[didx.at[c]], add=True)
                return carry

            lax.fori_loop(0, _IBLK, pipe, 0)

        plsc.subcore_barrier()

        base = sid * rows_per_tile
        pltpu.sync_copy(acc.at[pl.ds(base, rows_per_tile)],
                        out_hbm.at[cid, pl.ds(base, rows_per_tile)])

    return spmm


@functools.cache
def _make_mlp(N, D, BLK=1000):
    def body(eps_ref, x_ref, p0_ref, p1_ref, wa_ref, ba_ref, wb_ref, bb_ref, o_ref):
        scale = 1.0 + eps_ref[0]
        hin = x_ref[:] * scale + p0_ref[:] + p1_ref[:]
        t = lax.dot_general(hin, wa_ref[:], (((1,), (1,)), ((), ())),
                            preferred_element_type=jnp.float32)
        t = jnp.maximum(t + ba_ref[:], 0.0)
        o = lax.dot_general(t, wb_ref[:], (((1,), (1,)), ((), ())),
                            preferred_element_type=jnp.float32)
        o_ref[:] = o + bb_ref[:]

    return pl.pallas_call(
        body,
        grid=(N // BLK,),
        in_specs=[
            pl.BlockSpec(memory_space=pltpu.SMEM),
            pl.BlockSpec((BLK, D), lambda i: (i, 0)),
            pl.BlockSpec((BLK, D), lambda i: (i, 0)),
            pl.BlockSpec((BLK, D), lambda i: (i, 0)),
            pl.BlockSpec((D, D), lambda i: (0, 0)),
            pl.BlockSpec((1, D), lambda i: (0, 0)),
            pl.BlockSpec((D, D), lambda i: (0, 0)),
            pl.BlockSpec((1, D), lambda i: (0, 0)),
        ],
        out_specs=pl.BlockSpec((BLK, D), lambda i: (i, 0)),
        out_shape=jax.ShapeDtypeStruct((N, D), jnp.float32),
    )


def kernel(x, edge_index, eps, W1a, b1a, W1b, b1b, W2a, b2a, W2b, b2b):
    N, D = x.shape
    E = edge_index.shape[1]
    info = plsc.get_sparse_core_info()
    NW = info.num_cores * info.num_subcores
    per_t_pad = _NBLK * _IBLK * _CHUNK
    pad = per_t_pad - E // NW
    N_pad = -(-N // (8 * info.num_subcores)) * (8 * info.num_subcores)
    pad_src = jnp.zeros((NW, pad), jnp.int32)
    pad_dst = N + jnp.broadcast_to(
        jnp.arange(pad, dtype=jnp.int32) % max(N_pad - N, 1), (NW, pad))
    src = jnp.concatenate(
        [edge_index[0].astype(jnp.int32).reshape(NW, E // NW), pad_src], axis=1)
    dst = jnp.concatenate(
        [edge_index[1].astype(jnp.int32).reshape(NW, E // NW), pad_dst], axis=1)
    src = src.reshape(NW, _NBLK * _IBLK, _CHUNK)
    dst = dst.reshape(NW, _NBLK * _IBLK, _CHUNK)
    E_pad = NW * per_t_pad
    eps1 = jnp.asarray(eps, jnp.float32).reshape(1)

    spmm = _make_spmm(N, D, E_pad)
    mlp = _make_mlp(N, D)

    p = spmm(x, src, dst)
    h = mlp(eps1, x, p[0], p[1], W1a, b1a.reshape(1, D), W1b, b1b.reshape(1, D))
    p2 = spmm(h, src, dst)
    out = mlp(eps1, h, p2[0], p2[1], W2a, b2a.reshape(1, D), W2b, b2b.reshape(1, D))
    return out


# speedup vs baseline: 2.3040x; 2.3040x over previous
import functools

import jax
import jax.numpy as jnp
from jax import lax
from jax.experimental import pallas as pl
from jax.experimental.pallas import tpu as pltpu
from jax.experimental.pallas import tpu_sc as plsc

_CHUNK = 128
_IBLK = 80
_NBLK = 1


@functools.cache
def _make_spmm(N, D, E_pad):
    info = plsc.get_sparse_core_info()
    NC, NS = info.num_cores, info.num_subcores
    NW = NC * NS
    assert E_pad == NW * _NBLK * _IBLK * _CHUNK
    ZROWS = 8
    N_pad = -(-N // (ZROWS * NS)) * (ZROWS * NS)
    rows_per_tile = N_pad // NS
    n_z = rows_per_tile // ZROWS

    mesh = plsc.VectorSubcoreMesh(core_axis_name="c", subcore_axis_name="s")

    @functools.partial(
        pl.kernel,
        mesh=mesh,
        out_type=jax.ShapeDtypeStruct((NC, N_pad, D), jnp.float32),
        scratch_types=[
            pltpu.VMEM((_IBLK, _CHUNK), jnp.int32),
            pltpu.VMEM((_IBLK, _CHUNK), jnp.int32),
            pltpu.VMEM((_CHUNK, D), jnp.float32),
            pltpu.VMEM((ZROWS, D), jnp.float32),
            pltpu.VMEM_SHARED((N_pad, D), jnp.float32),
            pltpu.SemaphoreType.DMA,
            pltpu.SemaphoreType.DMA,
        ],
    )
    def spmm(x_hbm, src_hbm, dst_hbm, out_hbm, sidx, didx, rows,
             zbuf, acc, gsem, ssem):
        cid = lax.axis_index("c")
        sid = lax.axis_index("s")
        wid = sid * NC + cid

        for i in range(ZROWS):
            for j in range(D // 16):
                zbuf[i, pl.ds(j * 16, 16)] = jnp.zeros((16,), jnp.float32)

        def zacc(k, carry):
            pltpu.sync_copy(zbuf, acc.at[pl.ds(sid * rows_per_tile + k * ZROWS, ZROWS)])
            return carry

        lax.fori_loop(0, n_z, zacc, 0)
        plsc.subcore_barrier()

        def g_start(c):
            pltpu.make_async_copy(x_hbm.at[sidx.at[c]], rows, gsem).start()

        def g_wait(c):
            pltpu.make_async_copy(x_hbm.at[sidx.at[c]], rows, gsem).wait()

        def s_start(c):
            pltpu.async_copy(rows, acc.at[didx.at[c]], ssem, add=True)

        def s_wait(c):
            pltpu.make_async_copy(rows, acc.at[didx.at[c]], ssem).wait()

        for r in range(_NBLK):
            pltpu.sync_copy(src_hbm.at[wid, pl.ds(r * _IBLK, _IBLK)], sidx)
            pltpu.sync_copy(dst_hbm.at[wid, pl.ds(r * _IBLK, _IBLK)], didx)
            def pipe(c, carry):
                g_start(c)
                g_wait(c)
                pltpu.sync_copy(rows, acc.at[didx.at[c]], add=True)
                return carry

            lax.fori_loop(0, _IBLK, pipe, 0)

        plsc.subcore_barrier()

        base = sid * rows_per_tile
        pltpu.sync_copy(acc.at[pl.ds(base, rows_per_tile)],
                        out_hbm.at[cid, pl.ds(base, rows_per_tile)])

    return spmm


@functools.cache
def _make_mlp(N, D, BLK=1000):
    def body(eps_ref, x_ref, p0_ref, p1_ref, wa_ref, ba_ref, wb_ref, bb_ref, o_ref):
        scale = 1.0 + eps_ref[0]
        hin = x_ref[:] * scale + p0_ref[:] + p1_ref[:]
        t = lax.dot_general(hin, wa_ref[:], (((1,), (1,)), ((), ())),
                            preferred_element_type=jnp.float32)
        t = jnp.maximum(t + ba_ref[:], 0.0)
        o = lax.dot_general(t, wb_ref[:], (((1,), (1,)), ((), ())),
                            preferred_element_type=jnp.float32)
        o_ref[:] = o + bb_ref[:]

    return pl.pallas_call(
        body,
        grid=(N // BLK,),
        in_specs=[
            pl.BlockSpec(memory_space=pltpu.SMEM),
            pl.BlockSpec((BLK, D), lambda i: (i, 0)),
            pl.BlockSpec((BLK, D), lambda i: (i, 0)),
            pl.BlockSpec((BLK, D), lambda i: (i, 0)),
            pl.BlockSpec((D, D), lambda i: (0, 0)),
            pl.BlockSpec((1, D), lambda i: (0, 0)),
            pl.BlockSpec((D, D), lambda i: (0, 0)),
            pl.BlockSpec((1, D), lambda i: (0, 0)),
        ],
        out_specs=pl.BlockSpec((BLK, D), lambda i: (i, 0)),
        out_shape=jax.ShapeDtypeStruct((N, D), jnp.float32),
    )


def kernel(x, edge_index, eps, W1a, b1a, W1b, b1b, W2a, b2a, W2b, b2b):
    N, D = x.shape
    E = edge_index.shape[1]
    info = plsc.get_sparse_core_info()
    NW = info.num_cores * info.num_subcores
    per_t_pad = _NBLK * _IBLK * _CHUNK
    pad = per_t_pad - E // NW
    N_pad = -(-N // (8 * info.num_subcores)) * (8 * info.num_subcores)
    pad_src = ((jnp.arange(NW, dtype=jnp.int32)[:, None] * 331
                + jnp.arange(pad, dtype=jnp.int32)[None, :] * 13) % N)
    pad_dst = N + jnp.broadcast_to(
        jnp.arange(pad, dtype=jnp.int32) % max(N_pad - N, 1), (NW, pad))
    src = jnp.concatenate(
        [edge_index[0].astype(jnp.int32).reshape(NW, E // NW), pad_src], axis=1)
    dst = jnp.concatenate(
        [edge_index[1].astype(jnp.int32).reshape(NW, E // NW), pad_dst], axis=1)
    src = src.reshape(NW, _NBLK * _IBLK, _CHUNK)
    dst = dst.reshape(NW, _NBLK * _IBLK, _CHUNK)
    E_pad = NW * per_t_pad
    eps1 = jnp.asarray(eps, jnp.float32).reshape(1)

    spmm = _make_spmm(N, D, E_pad)
    mlp = _make_mlp(N, D)

    p = spmm(x, src, dst)
    h = mlp(eps1, x, p[0], p[1], W1a, b1a.reshape(1, D), W1b, b1b.reshape(1, D))
    p2 = spmm(h, src, dst)
    out = mlp(eps1, h, p2[0], p2[1], W2a, b2a.reshape(1, D), W2b, b2b.reshape(1, D))
    return out


# speedup vs baseline: 2.8959x; 1.2569x over previous
import functools

import jax
import jax.numpy as jnp
from jax import lax
from jax.experimental import pallas as pl
from jax.experimental.pallas import tpu as pltpu
from jax.experimental.pallas import tpu_sc as plsc

_CHUNK = 128
_IBLK = 40
_NBLK = 2


@functools.cache
def _make_spmm(N, D, E_pad):
    info = plsc.get_sparse_core_info()
    NC, NS = info.num_cores, info.num_subcores
    NW = NC * NS
    assert E_pad == NW * _NBLK * _IBLK * _CHUNK
    ZROWS = 8
    N_pad = -(-N // (ZROWS * NS)) * (ZROWS * NS)
    rows_per_tile = N_pad // NS
    n_z = rows_per_tile // ZROWS

    mesh = plsc.VectorSubcoreMesh(core_axis_name="c", subcore_axis_name="s")

    @functools.partial(
        pl.kernel,
        mesh=mesh,
        out_type=jax.ShapeDtypeStruct((NC, N_pad, D), jnp.float32),
        scratch_types=[
            pltpu.VMEM((_IBLK, _CHUNK), jnp.int32),
            pltpu.VMEM((_IBLK, _CHUNK), jnp.int32),
            pltpu.VMEM((2, _CHUNK, D), jnp.float32),
            pltpu.VMEM((ZROWS, D), jnp.float32),
            pltpu.VMEM_SHARED((N_pad, D), jnp.float32),
            pltpu.SemaphoreType.DMA,
            pltpu.SemaphoreType.DMA,
        ],
    )
    def spmm(x_hbm, src_hbm, dst_hbm, out_hbm, sidx, didx, rows,
             zbuf, acc, gsem, ssem):
        cid = lax.axis_index("c")
        sid = lax.axis_index("s")
        wid = sid * NC + cid

        for i in range(ZROWS):
            for j in range(D // 16):
                zbuf[i, pl.ds(j * 16, 16)] = jnp.zeros((16,), jnp.float32)

        def zacc(k, carry):
            pltpu.sync_copy(zbuf, acc.at[pl.ds(sid * rows_per_tile + k * ZROWS, ZROWS)])
            return carry

        lax.fori_loop(0, n_z, zacc, 0)
        plsc.subcore_barrier()

        def g_start(c):
            pltpu.make_async_copy(x_hbm.at[sidx.at[c]], rows.at[c % 2], gsem).start()

        def g_wait(c):
            pltpu.make_async_copy(x_hbm.at[sidx.at[c]], rows.at[c % 2], gsem).wait()

        def s_start(c):
            pltpu.async_copy(rows.at[c % 2], acc.at[didx.at[c]], ssem, add=True)

        def s_wait(c):
            pltpu.make_async_copy(rows.at[c % 2], acc.at[didx.at[c]], ssem).wait()

        for r in range(_NBLK):
            pltpu.sync_copy(src_hbm.at[wid, pl.ds(r * _IBLK, _IBLK)], sidx)
            pltpu.sync_copy(dst_hbm.at[wid, pl.ds(r * _IBLK, _IBLK)], didx)
            g_start(0)

            def pipe(c, carry):
                g_wait(c)

                @pl.when(c < _IBLK - 1)
                def _prefetch():
                    @pl.when(c >= 1)
                    def _free():
                        s_wait(c - 1)

                    g_start(c + 1)

                s_start(c)
                return carry

            lax.fori_loop(0, _IBLK, pipe, 0)
            s_wait(_IBLK - 2)
            s_wait(_IBLK - 1)

        plsc.subcore_barrier()

        base = sid * rows_per_tile
        pltpu.sync_copy(acc.at[pl.ds(base, rows_per_tile)],
                        out_hbm.at[cid, pl.ds(base, rows_per_tile)])

    return spmm


@functools.cache
def _make_mlp(N, D, BLK=1000):
    def body(eps_ref, x_ref, p0_ref, p1_ref, wa_ref, ba_ref, wb_ref, bb_ref, o_ref):
        scale = 1.0 + eps_ref[0]
        hin = x_ref[:] * scale + p0_ref[:] + p1_ref[:]
        t = lax.dot_general(hin, wa_ref[:], (((1,), (1,)), ((), ())),
                            preferred_element_type=jnp.float32)
        t = jnp.maximum(t + ba_ref[:], 0.0)
        o = lax.dot_general(t, wb_ref[:], (((1,), (1,)), ((), ())),
                            preferred_element_type=jnp.float32)
        o_ref[:] = o + bb_ref[:]

    return pl.pallas_call(
        body,
        grid=(N // BLK,),
        in_specs=[
            pl.BlockSpec(memory_space=pltpu.SMEM),
            pl.BlockSpec((BLK, D), lambda i: (i, 0)),
            pl.BlockSpec((BLK, D), lambda i: (i, 0)),
            pl.BlockSpec((BLK, D), lambda i: (i, 0)),
            pl.BlockSpec((D, D), lambda i: (0, 0)),
            pl.BlockSpec((1, D), lambda i: (0, 0)),
            pl.BlockSpec((D, D), lambda i: (0, 0)),
            pl.BlockSpec((1, D), lambda i: (0, 0)),
        ],
        out_specs=pl.BlockSpec((BLK, D), lambda i: (i, 0)),
        out_shape=jax.ShapeDtypeStruct((N, D), jnp.float32),
    )


def kernel(x, edge_index, eps, W1a, b1a, W1b, b1b, W2a, b2a, W2b, b2b):
    N, D = x.shape
    E = edge_index.shape[1]
    info = plsc.get_sparse_core_info()
    NW = info.num_cores * info.num_subcores
    per_t_pad = _NBLK * _IBLK * _CHUNK
    pad = per_t_pad - E // NW
    N_pad = -(-N // (8 * info.num_subcores)) * (8 * info.num_subcores)
    pad_src = ((jnp.arange(NW, dtype=jnp.int32)[:, None] * 331
                + jnp.arange(pad, dtype=jnp.int32)[None, :] * 13) % N)
    pad_dst = N + jnp.broadcast_to(
        jnp.arange(pad, dtype=jnp.int32) % max(N_pad - N, 1), (NW, pad))
    src = jnp.concatenate(
        [edge_index[0].astype(jnp.int32).reshape(NW, E // NW), pad_src], axis=1)
    dst = jnp.concatenate(
        [edge_index[1].astype(jnp.int32).reshape(NW, E // NW), pad_dst], axis=1)
    src = src.reshape(NW, _NBLK * _IBLK, _CHUNK)
    dst = dst.reshape(NW, _NBLK * _IBLK, _CHUNK)
    E_pad = NW * per_t_pad
    eps1 = jnp.asarray(eps, jnp.float32).reshape(1)

    spmm = _make_spmm(N, D, E_pad)
    mlp = _make_mlp(N, D)

    p = spmm(x, src, dst)
    h = mlp(eps1, x, p[0], p[1], W1a, b1a.reshape(1, D), W1b, b1b.reshape(1, D))
    p2 = spmm(h, src, dst)
    out = mlp(eps1, h, p2[0], p2[1], W2a, b2a.reshape(1, D), W2b, b2b.reshape(1, D))
    return out


# speedup vs baseline: 2.9184x; 1.0078x over previous
import functools

import jax
import jax.numpy as jnp
from jax import lax
from jax.experimental import pallas as pl
from jax.experimental.pallas import tpu as pltpu
from jax.experimental.pallas import tpu_sc as plsc

_CHUNK = 128
_IBLK = 40
_NBLK = 2


@functools.cache
def _make_spmm(N, D, E_pad):
    info = plsc.get_sparse_core_info()
    NC, NS = info.num_cores, info.num_subcores
    NW = NC * NS
    assert E_pad == NW * _NBLK * _IBLK * _CHUNK
    ZROWS = 8
    N_pad = -(-N // (ZROWS * NS)) * (ZROWS * NS)
    rows_per_tile = N_pad // NS
    n_z = rows_per_tile // ZROWS

    mesh = plsc.VectorSubcoreMesh(core_axis_name="c", subcore_axis_name="s")

    @functools.partial(
        pl.kernel,
        mesh=mesh,
        out_type=jax.ShapeDtypeStruct((NC, N_pad, D), jnp.float32),
        scratch_types=[
            pltpu.VMEM((_IBLK, _CHUNK), jnp.int32),
            pltpu.VMEM((_NBLK * _IBLK, _CHUNK), jnp.int32),
            pltpu.VMEM((2, _CHUNK, D), jnp.float32),
            pltpu.VMEM((ZROWS, D), jnp.float32),
            pltpu.VMEM_SHARED((N_pad, D), jnp.float32),
            pltpu.SemaphoreType.DMA,
            pltpu.SemaphoreType.DMA,
        ],
    )
    def spmm(x_hbm, src_hbm, dst_hbm, out_hbm, sidx, didx, rows,
             zbuf, acc, gsem, ssem):
        cid = lax.axis_index("c")
        sid = lax.axis_index("s")
        wid = sid * NC + cid

        for i in range(ZROWS):
            for j in range(D // 16):
                zbuf[i, pl.ds(j * 16, 16)] = jnp.zeros((16,), jnp.float32)

        def zacc(k, carry):
            pltpu.sync_copy(zbuf, acc.at[pl.ds(sid * rows_per_tile + k * ZROWS, ZROWS)])
            return carry

        lax.fori_loop(0, n_z, zacc, 0)
        plsc.subcore_barrier()

        def g_start(c):
            pltpu.make_async_copy(
                x_hbm.at[sidx.at[c % _IBLK]], rows.at[c % 2], gsem).start()

        def g_wait(c):
            pltpu.make_async_copy(
                x_hbm.at[sidx.at[c % _IBLK]], rows.at[c % 2], gsem).wait()

        def s_start(c):
            pltpu.async_copy(rows.at[c % 2], acc.at[didx.at[c]], ssem, add=True)

        def s_wait(c):
            pltpu.make_async_copy(rows.at[c % 2], acc.at[didx.at[c]], ssem).wait()

        n_chunks = _NBLK * _IBLK
        pltpu.sync_copy(src_hbm.at[wid, pl.ds(0, _IBLK)], sidx)
        pltpu.sync_copy(dst_hbm.at[wid], didx)
        g_start(0)

        def pipe(c, carry):
            g_wait(c)

            @pl.when(c < n_chunks - 1)
            def _prefetch():
                @pl.when(c >= 1)
                def _free():
                    s_wait(c - 1)

                @pl.when(c % _IBLK == _IBLK - 1)
                def _refill():
                    pltpu.sync_copy(
                        src_hbm.at[wid, pl.ds((c // _IBLK + 1) * _IBLK, _IBLK)],
                        sidx)

                g_start(c + 1)

            s_start(c)
            return carry

        lax.fori_loop(0, n_chunks, pipe, 0)
        s_wait(n_chunks - 2)
        s_wait(n_chunks - 1)
        plsc.subcore_barrier()

        base = sid * rows_per_tile
        pltpu.sync_copy(acc.at[pl.ds(base, rows_per_tile)],
                        out_hbm.at[cid, pl.ds(base, rows_per_tile)])

    return spmm


@functools.cache
def _make_mlp(N, D, BLK=1000):
    def body(eps_ref, x_ref, p0_ref, p1_ref, wa_ref, ba_ref, wb_ref, bb_ref, o_ref):
        scale = 1.0 + eps_ref[0]
        hin = x_ref[:] * scale + p0_ref[:] + p1_ref[:]
        t = lax.dot_general(hin, wa_ref[:], (((1,), (1,)), ((), ())),
                            preferred_element_type=jnp.float32)
        t = jnp.maximum(t + ba_ref[:], 0.0)
        o = lax.dot_general(t, wb_ref[:], (((1,), (1,)), ((), ())),
                            preferred_element_type=jnp.float32)
        o_ref[:] = o + bb_ref[:]

    return pl.pallas_call(
        body,
        grid=(N // BLK,),
        in_specs=[
            pl.BlockSpec(memory_space=pltpu.SMEM),
            pl.BlockSpec((BLK, D), lambda i: (i, 0)),
            pl.BlockSpec((BLK, D), lambda i: (i, 0)),
            pl.BlockSpec((BLK, D), lambda i: (i, 0)),
            pl.BlockSpec((D, D), lambda i: (0, 0)),
            pl.BlockSpec((1, D), lambda i: (0, 0)),
            pl.BlockSpec((D, D), lambda i: (0, 0)),
            pl.BlockSpec((1, D), lambda i: (0, 0)),
        ],
        out_specs=pl.BlockSpec((BLK, D), lambda i: (i, 0)),
        out_shape=jax.ShapeDtypeStruct((N, D), jnp.float32),
    )


def kernel(x, edge_index, eps, W1a, b1a, W1b, b1b, W2a, b2a, W2b, b2b):
    N, D = x.shape
    E = edge_index.shape[1]
    info = plsc.get_sparse_core_info()
    NW = info.num_cores * info.num_subcores
    per_t_pad = _NBLK * _IBLK * _CHUNK
    pad = per_t_pad - E // NW
    N_pad = -(-N // (8 * info.num_subcores)) * (8 * info.num_subcores)
    pad_src = ((jnp.arange(NW, dtype=jnp.int32)[:, None] * 331
                + jnp.arange(pad, dtype=jnp.int32)[None, :] * 13) % N)
    pad_dst = N + jnp.broadcast_to(
        jnp.arange(pad, dtype=jnp.int32) % max(N_pad - N, 1), (NW, pad))
    src = jnp.concatenate(
        [edge_index[0].astype(jnp.int32).reshape(NW, E // NW), pad_src], axis=1)
    dst = jnp.concatenate(
        [edge_index[1].astype(jnp.int32).reshape(NW, E // NW), pad_dst], axis=1)
    src = src.reshape(NW, _NBLK * _IBLK, _CHUNK)
    dst = dst.reshape(NW, _NBLK * _IBLK, _CHUNK)
    E_pad = NW * per_t_pad
    eps1 = jnp.asarray(eps, jnp.float32).reshape(1)

    spmm = _make_spmm(N, D, E_pad)
    mlp = _make_mlp(N, D)

    p = spmm(x, src, dst)
    h = mlp(eps1, x, p[0], p[1], W1a, b1a.reshape(1, D), W1b, b1b.reshape(1, D))
    p2 = spmm(h, src, dst)
    out = mlp(eps1, h, p2[0], p2[1], W2a, b2a.reshape(1, D), W2b, b2b.reshape(1, D))
    return out


# speedup vs baseline: 3.0672x; 1.0510x over previous
import functools

import jax
import jax.numpy as jnp
from jax import lax
from jax.experimental import pallas as pl
from jax.experimental.pallas import tpu as pltpu
from jax.experimental.pallas import tpu_sc as plsc

_CHUNK = 128
_IBLK = 40
_NBLK = 2


@functools.cache
def _make_spmm(N, D, E_pad):
    info = plsc.get_sparse_core_info()
    NC, NS = info.num_cores, info.num_subcores
    NW = NC * NS
    assert E_pad == NW * _NBLK * _IBLK * _CHUNK
    ZROWS = 8
    N_pad = -(-N // (ZROWS * NS)) * (ZROWS * NS)
    rows_per_tile = N_pad // NS
    n_z = rows_per_tile // ZROWS

    mesh = plsc.VectorSubcoreMesh(core_axis_name="c", subcore_axis_name="s")

    @functools.partial(
        pl.kernel,
        mesh=mesh,
        out_type=jax.ShapeDtypeStruct((NC, N_pad, D), jnp.float32),
        scratch_types=[
            pltpu.VMEM((_IBLK, _CHUNK), jnp.int32),
            pltpu.VMEM((_NBLK * _IBLK, _CHUNK), jnp.int32),
            pltpu.VMEM((2, _CHUNK, D), jnp.float32),
            pltpu.VMEM((ZROWS, D), jnp.float32),
            pltpu.VMEM_SHARED((N_pad, D), jnp.float32),
            pltpu.SemaphoreType.DMA,
            pltpu.SemaphoreType.DMA,
        ],
    )
    def spmm(x_hbm, src_hbm, dst_hbm, out_hbm, sidx, didx, rows,
             zbuf, acc, gsem, ssem):
        cid = lax.axis_index("c")
        sid = lax.axis_index("s")
        wid = sid * NC + cid

        for i in range(ZROWS):
            for j in range(D // 16):
                zbuf[i, pl.ds(j * 16, 16)] = jnp.zeros((16,), jnp.float32)

        def zdesc(k):
            return pltpu.make_async_copy(
                zbuf, acc.at[pl.ds(sid * rows_per_tile + k * ZROWS, ZROWS)], ssem)

        def zfire(k, carry):
            zdesc(k).start()
            return carry

        def zdrain(k, carry):
            zdesc(k).wait()
            return carry

        lax.fori_loop(0, n_z, zfire, 0)

        def g_start(c):
            pltpu.make_async_copy(
                x_hbm.at[sidx.at[c % _IBLK]], rows.at[c % 2], gsem).start()

        def g_wait(c):
            pltpu.make_async_copy(
                x_hbm.at[sidx.at[c % _IBLK]], rows.at[c % 2], gsem).wait()

        def s_start(c):
            pltpu.async_copy(rows.at[c % 2], acc.at[didx.at[c]], ssem, add=True)

        def s_wait(c):
            pltpu.make_async_copy(rows.at[c % 2], acc.at[didx.at[c]], ssem).wait()

        n_chunks = _NBLK * _IBLK
        pltpu.sync_copy(src_hbm.at[wid, pl.ds(0, _IBLK)], sidx)
        pltpu.sync_copy(dst_hbm.at[wid], didx)
        g_start(0)
        lax.fori_loop(0, n_z, zdrain, 0)
        plsc.subcore_barrier()

        def pipe(c, carry):
            g_wait(c)

            @pl.when(c < n_chunks - 1)
            def _prefetch():
                @pl.when(c >= 1)
                def _free():
                    s_wait(c - 1)

                @pl.when(c % _IBLK == _IBLK - 1)
                def _refill():
                    pltpu.sync_copy(
                        src_hbm.at[wid, pl.ds((c // _IBLK + 1) * _IBLK, _IBLK)],
                        sidx)

                g_start(c + 1)

            s_start(c)
            return carry

        lax.fori_loop(0, n_chunks, pipe, 0)
        s_wait(n_chunks - 2)
        s_wait(n_chunks - 1)
        plsc.subcore_barrier()

        base = sid * rows_per_tile
        pltpu.sync_copy(acc.at[pl.ds(base, rows_per_tile)],
                        out_hbm.at[cid, pl.ds(base, rows_per_tile)])

    return spmm


@functools.cache
def _make_mlp(N, D, BLK=2000):
    def body(eps_ref, x_ref, p0_ref, p1_ref, wa_ref, ba_ref, wb_ref, bb_ref, o_ref):
        scale = 1.0 + eps_ref[0]
        hin = x_ref[:] * scale + p0_ref[:] + p1_ref[:]
        t = lax.dot_general(hin, wa_ref[:], (((1,), (1,)), ((), ())),
                            preferred_element_type=jnp.float32)
        t = jnp.maximum(t + ba_ref[:], 0.0)
        o = lax.dot_general(t, wb_ref[:], (((1,), (1,)), ((), ())),
                            preferred_element_type=jnp.float32)
        o_ref[:] = o + bb_ref[:]

    return pl.pallas_call(
        body,
        grid=(N // BLK,),
        in_specs=[
            pl.BlockSpec(memory_space=pltpu.SMEM),
            pl.BlockSpec((BLK, D), lambda i: (i, 0)),
            pl.BlockSpec((BLK, D), lambda i: (i, 0)),
            pl.BlockSpec((BLK, D), lambda i: (i, 0)),
            pl.BlockSpec((D, D), lambda i: (0, 0)),
            pl.BlockSpec((1, D), lambda i: (0, 0)),
            pl.BlockSpec((D, D), lambda i: (0, 0)),
            pl.BlockSpec((1, D), lambda i: (0, 0)),
        ],
        out_specs=pl.BlockSpec((BLK, D), lambda i: (i, 0)),
        out_shape=jax.ShapeDtypeStruct((N, D), jnp.float32),
    )


def kernel(x, edge_index, eps, W1a, b1a, W1b, b1b, W2a, b2a, W2b, b2b):
    N, D = x.shape
    E = edge_index.shape[1]
    info = plsc.get_sparse_core_info()
    NW = info.num_cores * info.num_subcores
    per_t_pad = _NBLK * _IBLK * _CHUNK
    pad = per_t_pad - E // NW
    N_pad = -(-N // (8 * info.num_subcores)) * (8 * info.num_subcores)
    pad_src = ((jnp.arange(NW, dtype=jnp.int32)[:, None] * 331
                + jnp.arange(pad, dtype=jnp.int32)[None, :] * 13) % N)
    pad_dst = N + jnp.broadcast_to(
        jnp.arange(pad, dtype=jnp.int32) % max(N_pad - N, 1), (NW, pad))
    src = jnp.concatenate(
        [edge_index[0].astype(jnp.int32).reshape(NW, E // NW), pad_src], axis=1)
    dst = jnp.concatenate(
        [edge_index[1].astype(jnp.int32).reshape(NW, E // NW), pad_dst], axis=1)
    src = src.reshape(NW, _NBLK * _IBLK, _CHUNK)
    dst = dst.reshape(NW, _NBLK * _IBLK, _CHUNK)
    E_pad = NW * per_t_pad
    eps1 = jnp.asarray(eps, jnp.float32).reshape(1)

    spmm = _make_spmm(N, D, E_pad)
    mlp = _make_mlp(N, D)

    p = spmm(x, src, dst)
    h = mlp(eps1, x, p[0], p[1], W1a, b1a.reshape(1, D), W1b, b1b.reshape(1, D))
    p2 = spmm(h, src, dst)
    out = mlp(eps1, h, p2[0], p2[1], W2a, b2a.reshape(1, D), W2b, b2b.reshape(1, D))
    return out


# speedup vs baseline: 3.2097x; 1.0464x over previous
import functools

import jax
import jax.numpy as jnp
from jax import lax
from jax.experimental import pallas as pl
from jax.experimental.pallas import tpu as pltpu
from jax.experimental.pallas import tpu_sc as plsc

_CHUNK = 128
_IBLK = 40
_NBLK = 2


@functools.cache
def _make_spmm(N, D, E_pad):
    info = plsc.get_sparse_core_info()
    NC, NS = info.num_cores, info.num_subcores
    NW = NC * NS
    assert E_pad == NW * _NBLK * _IBLK * _CHUNK
    ZROWS = 8
    N_pad = -(-N // (ZROWS * NS)) * (ZROWS * NS)
    rows_per_tile = N_pad // NS
    n_z = rows_per_tile // ZROWS

    mesh = plsc.VectorSubcoreMesh(core_axis_name="c", subcore_axis_name="s")

    @functools.partial(
        pl.kernel,
        mesh=mesh,
        out_type=jax.ShapeDtypeStruct((NC, N_pad, D), jnp.float32),
        scratch_types=[
            pltpu.VMEM((_IBLK, _CHUNK), jnp.int32),
            pltpu.VMEM((_NBLK * _IBLK, _CHUNK), jnp.int32),
            pltpu.VMEM((2, _CHUNK, D), jnp.float32),
            pltpu.VMEM((ZROWS, D), jnp.float32),
            pltpu.VMEM_SHARED((N_pad, D), jnp.float32),
            pltpu.SemaphoreType.DMA,
            pltpu.SemaphoreType.DMA,
        ],
    )
    def spmm(x_hbm, src_hbm, dst_hbm, out_hbm, sidx, didx, rows,
             zbuf, acc, gsem, ssem):
        cid = lax.axis_index("c")
        sid = lax.axis_index("s")
        wid = sid * NC + cid

        for i in range(ZROWS):
            for j in range(D // 16):
                zbuf[i, pl.ds(j * 16, 16)] = jnp.zeros((16,), jnp.float32)

        def zdesc(k):
            return pltpu.make_async_copy(
                zbuf, acc.at[pl.ds(sid * rows_per_tile + k * ZROWS, ZROWS)], ssem)

        def zfire(k, carry):
            zdesc(k).start()
            return carry

        def zdrain(k, carry):
            zdesc(k).wait()
            return carry

        lax.fori_loop(0, n_z, zfire, 0)

        def g_start(c):
            pltpu.make_async_copy(
                x_hbm.at[sidx.at[c % _IBLK]], rows.at[c % 2], gsem).start()

        def g_wait(c):
            pltpu.make_async_copy(
                x_hbm.at[sidx.at[c % _IBLK]], rows.at[c % 2], gsem).wait()

        def s_start(c):
            pltpu.async_copy(rows.at[c % 2], acc.at[didx.at[c]], ssem, add=True)

        def s_wait(c):
            pltpu.make_async_copy(rows.at[c % 2], acc.at[didx.at[c]], ssem).wait()

        n_chunks = _NBLK * _IBLK
        pltpu.sync_copy(src_hbm.at[wid, pl.ds(0, _IBLK)], sidx)
        pltpu.sync_copy(dst_hbm.at[wid], didx)
        g_start(0)
        lax.fori_loop(0, n_z, zdrain, 0)
        plsc.subcore_barrier()

        def pipe(c, carry):
            g_wait(c)

            @pl.when(c < n_chunks - 1)
            def _prefetch():
                @pl.when(c >= 1)
                def _free():
                    s_wait(c - 1)

                @pl.when(c % _IBLK == _IBLK - 1)
                def _refill():
                    pltpu.sync_copy(
                        src_hbm.at[wid, pl.ds((c // _IBLK + 1) * _IBLK, _IBLK)],
                        sidx)

                g_start(c + 1)

            s_start(c)
            return carry

        lax.fori_loop(0, n_chunks, pipe, 0)
        s_wait(n_chunks - 2)
        s_wait(n_chunks - 1)
        plsc.subcore_barrier()

        base = sid * rows_per_tile
        pltpu.sync_copy(acc.at[pl.ds(base, rows_per_tile)],
                        out_hbm.at[cid, pl.ds(base, rows_per_tile)])

    return spmm


@functools.cache
def _make_mlp(N, D, BLK=2000):
    def body(eps_ref, x_ref, p_ref, wa_ref, ba_ref, wb_ref, bb_ref, o_ref):
        scale = 1.0 + eps_ref[0]
        hin = x_ref[:] * scale + p_ref[0] + p_ref[1]
        t = lax.dot_general(hin, wa_ref[:], (((1,), (1,)), ((), ())),
                            preferred_element_type=jnp.float32)
        t = jnp.maximum(t + ba_ref[:], 0.0)
        o = lax.dot_general(t, wb_ref[:], (((1,), (1,)), ((), ())),
                            preferred_element_type=jnp.float32)
        o_ref[:] = o + bb_ref[:]

    return pl.pallas_call(
        body,
        grid=(N // BLK,),
        in_specs=[
            pl.BlockSpec(memory_space=pltpu.SMEM),
            pl.BlockSpec((BLK, D), lambda i: (i, 0)),
            pl.BlockSpec((2, BLK, D), lambda i: (0, i, 0)),
            pl.BlockSpec((D, D), lambda i: (0, 0)),
            pl.BlockSpec((1, D), lambda i: (0, 0)),
            pl.BlockSpec((D, D), lambda i: (0, 0)),
            pl.BlockSpec((1, D), lambda i: (0, 0)),
        ],
        out_specs=pl.BlockSpec((BLK, D), lambda i: (i, 0)),
        out_shape=jax.ShapeDtypeStruct((N, D), jnp.float32),
    )


def kernel(x, edge_index, eps, W1a, b1a, W1b, b1b, W2a, b2a, W2b, b2b):
    N, D = x.shape
    E = edge_index.shape[1]
    info = plsc.get_sparse_core_info()
    NW = info.num_cores * info.num_subcores
    per_t_pad = _NBLK * _IBLK * _CHUNK
    pad = per_t_pad - E // NW
    N_pad = -(-N // (8 * info.num_subcores)) * (8 * info.num_subcores)
    pad_src = ((jnp.arange(NW, dtype=jnp.int32)[:, None] * 331
                + jnp.arange(pad, dtype=jnp.int32)[None, :] * 13) % N)
    pad_dst = N + jnp.broadcast_to(
        jnp.arange(pad, dtype=jnp.int32) % max(N_pad - N, 1), (NW, pad))
    src = jnp.concatenate(
        [edge_index[0].astype(jnp.int32).reshape(NW, E // NW), pad_src], axis=1)
    dst = jnp.concatenate(
        [edge_index[1].astype(jnp.int32).reshape(NW, E // NW), pad_dst], axis=1)
    src = src.reshape(NW, _NBLK * _IBLK, _CHUNK)
    dst = dst.reshape(NW, _NBLK * _IBLK, _CHUNK)
    E_pad = NW * per_t_pad
    eps1 = jnp.asarray(eps, jnp.float32).reshape(1)

    spmm = _make_spmm(N, D, E_pad)
    mlp = _make_mlp(N, D)

    p = spmm(x, src, dst)
    h = mlp(eps1, x, p, W1a, b1a.reshape(1, D), W1b, b1b.reshape(1, D))
    p2 = spmm(h, src, dst)
    out = mlp(eps1, h, p2, W2a, b2a.reshape(1, D), W2b, b2b.reshape(1, D))
    return out


# speedup vs baseline: 3.2247x; 1.0047x over previous
import functools

import jax
import jax.numpy as jnp
from jax import lax
from jax.experimental import pallas as pl
from jax.experimental.pallas import tpu as pltpu
from jax.experimental.pallas import tpu_sc as plsc

_CHUNK = 128
_IBLK = 40
_NBLK = 2


@functools.cache
def _make_spmm(N, D, E_pad):
    info = plsc.get_sparse_core_info()
    NC, NS = info.num_cores, info.num_subcores
    NW = NC * NS
    assert E_pad == NW * _NBLK * _IBLK * _CHUNK
    ZROWS = 8
    N_pad = -(-N // (ZROWS * NS)) * (ZROWS * NS)
    rows_per_tile = N_pad // NS
    n_z = rows_per_tile // ZROWS

    mesh = plsc.VectorSubcoreMesh(core_axis_name="c", subcore_axis_name="s")

    @functools.partial(
        pl.kernel,
        mesh=mesh,
        out_type=jax.ShapeDtypeStruct((NC, N_pad, D), jnp.float32),
        scratch_types=[
            pltpu.VMEM((_IBLK, _CHUNK), jnp.int32),
            pltpu.VMEM((_NBLK * _IBLK, _CHUNK), jnp.int32),
            pltpu.VMEM((2, _CHUNK, D), jnp.float32),
            pltpu.VMEM((ZROWS, D), jnp.float32),
            pltpu.VMEM_SHARED((N_pad, D), jnp.float32),
            pltpu.SemaphoreType.DMA,
            pltpu.SemaphoreType.DMA,
        ],
    )
    def spmm(x_hbm, src_hbm, dst_hbm, out_hbm, sidx, didx, rows,
             zbuf, acc, gsem, ssem):
        cid = lax.axis_index("c")
        sid = lax.axis_index("s")
        wid = sid * NC + cid

        for i in range(ZROWS):
            for j in range(D // 16):
                zbuf[i, pl.ds(j * 16, 16)] = jnp.zeros((16,), jnp.float32)

        def zdesc(k):
            return pltpu.make_async_copy(
                zbuf, acc.at[pl.ds(sid * rows_per_tile + k * ZROWS, ZROWS)], ssem)

        def zfire(k, carry):
            zdesc(k).start()
            return carry

        def zdrain(k, carry):
            zdesc(k).wait()
            return carry

        lax.fori_loop(0, n_z, zfire, 0)

        def g_start(c):
            pltpu.make_async_copy(
                x_hbm.at[sidx.at[c % _IBLK]], rows.at[c % 2], gsem).start()

        def g_wait(c):
            pltpu.make_async_copy(
                x_hbm.at[sidx.at[c % _IBLK]], rows.at[c % 2], gsem).wait()

        def s_start(c):
            pltpu.async_copy(rows.at[c % 2], acc.at[didx.at[c]], ssem, add=True)

        def s_wait(c):
            pltpu.make_async_copy(rows.at[c % 2], acc.at[didx.at[c]], ssem).wait()

        n_chunks = _NBLK * _IBLK
        pltpu.sync_copy(src_hbm.at[wid, pl.ds(0, _IBLK)], sidx)
        pltpu.sync_copy(dst_hbm.at[wid], didx)
        g_start(0)
        lax.fori_loop(0, n_z, zdrain, 0)
        plsc.subcore_barrier()

        def pipe(c, carry):
            g_wait(c)

            @pl.when(c < n_chunks - 1)
            def _prefetch():
                @pl.when(c >= 1)
                def _free():
                    s_wait(c - 1)

                @pl.when(c % _IBLK == _IBLK - 1)
                def _refill():
                    pltpu.sync_copy(
                        src_hbm.at[wid, pl.ds((c // _IBLK + 1) * _IBLK, _IBLK)],
                        sidx)

                g_start(c + 1)

            s_start(c)
            return carry

        lax.fori_loop(0, n_chunks, pipe, 0)
        s_wait(n_chunks - 2)
        s_wait(n_chunks - 1)
        plsc.subcore_barrier()

        base = sid * rows_per_tile
        pltpu.sync_copy(acc.at[pl.ds(base, rows_per_tile)],
                        out_hbm.at[cid, pl.ds(base, rows_per_tile)])

    return spmm


@functools.cache
def _make_mlp(N, D, BLK=5000):
    def body(eps_ref, x_ref, p_ref, wa_ref, ba_ref, wb_ref, bb_ref, o_ref):
        scale = 1.0 + eps_ref[0]
        hin = x_ref[:] * scale + p_ref[0] + p_ref[1]
        t = lax.dot_general(hin, wa_ref[:], (((1,), (1,)), ((), ())),
                            preferred_element_type=jnp.float32)
        t = jnp.maximum(t + ba_ref[:], 0.0)
        o = lax.dot_general(t, wb_ref[:], (((1,), (1,)), ((), ())),
                            preferred_element_type=jnp.float32)
        o_ref[:] = o + bb_ref[:]

    return pl.pallas_call(
        body,
        grid=(N // BLK,),
        in_specs=[
            pl.BlockSpec(memory_space=pltpu.SMEM),
            pl.BlockSpec((BLK, D), lambda i: (i, 0)),
            pl.BlockSpec((2, BLK, D), lambda i: (0, i, 0)),
            pl.BlockSpec((D, D), lambda i: (0, 0)),
            pl.BlockSpec((1, D), lambda i: (0, 0)),
            pl.BlockSpec((D, D), lambda i: (0, 0)),
            pl.BlockSpec((1, D), lambda i: (0, 0)),
        ],
        out_specs=pl.BlockSpec((BLK, D), lambda i: (i, 0)),
        out_shape=jax.ShapeDtypeStruct((N, D), jnp.float32),
    )


def kernel(x, edge_index, eps, W1a, b1a, W1b, b1b, W2a, b2a, W2b, b2b):
    N, D = x.shape
    E = edge_index.shape[1]
    info = plsc.get_sparse_core_info()
    NW = info.num_cores * info.num_subcores
    per_t_pad = _NBLK * _IBLK * _CHUNK
    pad = per_t_pad - E // NW
    N_pad = -(-N // (8 * info.num_subcores)) * (8 * info.num_subcores)
    pad_src = ((jnp.arange(NW, dtype=jnp.int32)[:, None] * 331
                + jnp.arange(pad, dtype=jnp.int32)[None, :] * 13) % N)
    pad_dst = N + jnp.broadcast_to(
        jnp.arange(pad, dtype=jnp.int32) % max(N_pad - N, 1), (NW, pad))
    src = jnp.concatenate(
        [edge_index[0].astype(jnp.int32).reshape(NW, E // NW), pad_src], axis=1)
    dst = jnp.concatenate(
        [edge_index[1].astype(jnp.int32).reshape(NW, E // NW), pad_dst], axis=1)
    src = src.reshape(NW, _NBLK * _IBLK, _CHUNK)
    dst = dst.reshape(NW, _NBLK * _IBLK, _CHUNK)
    E_pad = NW * per_t_pad
    eps1 = jnp.asarray(eps, jnp.float32).reshape(1)

    spmm = _make_spmm(N, D, E_pad)
    mlp = _make_mlp(N, D)

    p = spmm(x, src, dst)
    h = mlp(eps1, x, p, W1a, b1a.reshape(1, D), W1b, b1b.reshape(1, D))
    p2 = spmm(h, src, dst)
    out = mlp(eps1, h, p2, W2a, b2a.reshape(1, D), W2b, b2b.reshape(1, D))
    return out
